# baseline (speedup 1.0000x reference)
import functools

import jax
import jax.numpy as jnp
from jax import lax
from jax.experimental import pallas as pl
from jax.experimental.pallas import tpu as pltpu
from jax.experimental.pallas import tpu_sc as plsc

_LS = 0.1
_BLK_W = 2048


def _sc_info():
    try:
        info = plsc.get_sparse_core_info()
        return info.num_cores, info.num_subcores
    except Exception:
        return 2, 16


def _gather_label_logits(pred, label):
    b, c = pred.shape
    flat = pred.reshape(b * c)
    nc, ns = _sc_info()
    nw = nc * ns
    bw = b // nw
    mesh = plsc.VectorSubcoreMesh(core_axis_name="c", subcore_axis_name="s")

    @functools.partial(
        pl.kernel,
        mesh=mesh,
        out_type=jax.ShapeDtypeStruct((b,), jnp.float32),
        scratch_types=[
            pltpu.VMEM((bw,), jnp.int32),
            pltpu.VMEM((bw,), jnp.int32),
            pltpu.VMEM((bw,), jnp.float32),
            pltpu.SemaphoreType.DMA,
        ],
    )
    def gather(pred_hbm, label_hbm, out_hbm, lab_v, idx_v, g_v, sem):
        wid = lax.axis_index("s") * nc + lax.axis_index("c")
        base = wid * bw
        pltpu.sync_copy(label_hbm.at[pl.ds(base, bw)], lab_v)
        for i in range(bw // 16):
            rows = base + i * 16 + lax.iota(jnp.int32, 16)
            idx_v[pl.ds(i * 16, 16)] = lab_v[pl.ds(i * 16, 16)] + rows * c
        pltpu.async_copy(pred_hbm.at[idx_v], g_v, sem).wait()
        pltpu.sync_copy(g_v, out_hbm.at[pl.ds(base, bw)])

    return gather(flat, label.astype(jnp.int32))


def _pass_body(c_total, g_ref, pred_ref, out_ref, s_acc, m_acc, z_acc):
    j = pl.program_id(0)
    nblk = pl.num_programs(0)
    blk_b, blk_w = pred_ref.shape

    @pl.when(j == 0)
    def _init():
        s_acc[...] = jnp.zeros_like(s_acc)
        m_acc[...] = jnp.full_like(m_acc, -jnp.inf)
        z_acc[...] = jnp.zeros_like(z_acc)

    @pl.when(j < nblk - 1)
    def _hot():
        x = pred_ref[...]
        s_acc[...] += jnp.sum(jnp.exp(x), axis=1, keepdims=True)
        m_acc[...] = jnp.maximum(m_acc[...], jnp.max(x, axis=1, keepdims=True))
        z_acc[...] += jnp.sum(jnp.where(x == 0.0, 1.0, 0.0), axis=1,
                              keepdims=True)

    @pl.when(j == nblk - 1)
    def _last():
        x = pred_ref[...]
        col = j * blk_w + lax.broadcasted_iota(jnp.int32, (blk_b, blk_w), 1)
        valid = col < c_total
        xm = jnp.where(valid, x, -jnp.inf)
        s = s_acc[...] + jnp.sum(jnp.exp(xm), axis=1, keepdims=True)
        m = jnp.maximum(m_acc[...], jnp.max(xm, axis=1, keepdims=True))
        z = z_acc[...] + jnp.sum(
            jnp.where(valid & (x == 0.0), 1.0, 0.0), axis=1, keepdims=True)
        g = g_ref[...]
        masked = jnp.exp(m) > 0.5 * s
        at_label = masked & (g == m)
        base = -(1.0 - _LS) * jnp.log(jnp.where(at_label, 1.0 - g, g))
        extra = jnp.where(masked & jnp.logical_not(at_label),
                          -_LS * jnp.log(1.0 - m), 0.0)
        loss = jnp.mean(base + extra)
        z_nonlabel = z - jnp.where(g == 0.0, 1.0, 0.0)
        has_nan = jnp.max(z_nonlabel) > 0.0
        out_ref[...] = jnp.full((1, 1),
                                jnp.where(has_nan, jnp.float32(jnp.nan), loss))


def _dense_pass(pred, g):
    b, c = pred.shape
    blk_w = min(_BLK_W, c)
    nblk = pl.cdiv(c, blk_w)
    out = pl.pallas_call(
        functools.partial(_pass_body, c),
        grid=(nblk,),
        in_specs=[
            pl.BlockSpec((b, 1), lambda j: (0, 0)),
            pl.BlockSpec((b, blk_w), lambda j: (0, j)),
        ],
        out_specs=pl.BlockSpec((1, 1), lambda j: (0, 0)),
        out_shape=jax.ShapeDtypeStruct((1, 1), jnp.float32),
        scratch_shapes=[
            pltpu.VMEM((b, 1), jnp.float32),
            pltpu.VMEM((b, 1), jnp.float32),
            pltpu.VMEM((b, 1), jnp.float32),
        ],
    )(g.reshape(b, 1), pred)
    return out.reshape(())


@jax.jit
def kernel(pred, label):
    g = _gather_label_logits(pred, label)
    return _dense_pass(pred, g)


# speedup vs baseline: 2.0910x; 2.0910x over previous
import functools

import jax
import jax.numpy as jnp
from jax import lax
from jax.experimental import pallas as pl
from jax.experimental.pallas import tpu as pltpu

_LS = 0.1
_BLK_W = 2048


def _pass_body(c_total, label_ref, pred_ref, out_ref, s_acc, m_acc, g_acc,
               z_acc):
    j = pl.program_id(0)
    nblk = pl.num_programs(0)
    blk_b, blk_w = pred_ref.shape

    @pl.when(j == 0)
    def _init():
        s_acc[...] = jnp.zeros_like(s_acc)
        m_acc[...] = jnp.full_like(m_acc, -jnp.inf)
        g_acc[...] = jnp.zeros_like(g_acc)
        z_acc[...] = jnp.zeros_like(z_acc)

    @pl.when(j < nblk - 1)
    def _hot():
        x = pred_ref[...]
        lab_local = label_ref[...] - j * blk_w
        match = lax.broadcasted_iota(jnp.int32, (blk_b, blk_w), 1) == lab_local
        g_acc[...] += jnp.sum(jnp.where(match, x, 0.0), axis=1, keepdims=True)
        s_acc[...] += jnp.sum(jnp.exp(x), axis=1, keepdims=True)
        m_acc[...] = jnp.maximum(m_acc[...], jnp.max(x, axis=1, keepdims=True))
        z_acc[...] += jnp.sum(jnp.where(x == 0.0, 1.0, 0.0), axis=1,
                              keepdims=True)

    @pl.when(j == nblk - 1)
    def _last():
        x = pred_ref[...]
        col = j * blk_w + lax.broadcasted_iota(jnp.int32, (blk_b, blk_w), 1)
        valid = col < c_total
        xm = jnp.where(valid, x, -jnp.inf)
        match = col == label_ref[...]
        g = g_acc[...] + jnp.sum(jnp.where(match, x, 0.0), axis=1,
                                 keepdims=True)
        s = s_acc[...] + jnp.sum(jnp.exp(xm), axis=1, keepdims=True)
        m = jnp.maximum(m_acc[...], jnp.max(xm, axis=1, keepdims=True))
        z = z_acc[...] + jnp.sum(
            jnp.where(valid & (x == 0.0), 1.0, 0.0), axis=1, keepdims=True)
        masked = jnp.exp(m) > 0.5 * s
        at_label = masked & (g == m)
        base = -(1.0 - _LS) * jnp.log(jnp.where(at_label, 1.0 - g, g))
        extra = jnp.where(masked & jnp.logical_not(at_label),
                          -_LS * jnp.log(1.0 - m), 0.0)
        loss = jnp.mean(base + extra)
        z_nonlabel = z - jnp.where(g == 0.0, 1.0, 0.0)
        has_nan = jnp.max(z_nonlabel) > 0.0
        out_ref[...] = jnp.full((1, 1),
                                jnp.where(has_nan, jnp.float32(jnp.nan), loss))


@functools.partial(jax.jit, static_argnames=("interpret",))
def kernel(pred, label, interpret=False):
    b, c = pred.shape
    blk_w = min(_BLK_W, c)
    nblk = pl.cdiv(c, blk_w)
    out = pl.pallas_call(
        functools.partial(_pass_body, c),
        grid=(nblk,),
        in_specs=[
            pl.BlockSpec((b, 1), lambda j: (0, 0)),
            pl.BlockSpec((b, blk_w), lambda j: (0, j)),
        ],
        out_specs=pl.BlockSpec((1, 1), lambda j: (0, 0)),
        out_shape=jax.ShapeDtypeStruct((1, 1), jnp.float32),
        scratch_shapes=[
            pltpu.VMEM((b, 1), jnp.float32),
            pltpu.VMEM((b, 1), jnp.float32),
            pltpu.VMEM((b, 1), jnp.float32),
            pltpu.VMEM((b, 1), jnp.float32),
        ],
        interpret=interpret,
    )(label.reshape(b, 1).astype(jnp.int32), pred)
    return out.reshape(())
